# Initial kernel scaffold; baseline (speedup 1.0000x reference)
import jax
import jax.numpy as jnp
from jax.experimental import pallas as pl


def kernel(x1, x2, x4, x5, codebook, W):
    raise NotImplementedError("write your pallas kernel here")


# speedup vs baseline: 1.0427x; 1.0427x over previous
import functools

import jax
import jax.numpy as jnp
from jax import lax
from jax.experimental import pallas as pl
from jax.experimental.pallas import tpu as pltpu
from jax.experimental.pallas import tpu_sc as plsc

V = 1000
D = 1000
DP = 1024
B = 4096
NC, NS = 2, 16
NW = NC * NS
BPW = B // NW
C = 16
NCH = BPW // C
LANES = 16


def _mm_body(a_ref, b_ref, o_ref):
    o_ref[...] = lax.dot_general(
        a_ref[...], b_ref[...],
        dimension_numbers=(((1,), (1,)), ((), ())),
        preferred_element_type=jnp.float32,
    )


def _make_table(codebook, w_pad):
    return pl.pallas_call(
        _mm_body,
        out_shape=jax.ShapeDtypeStruct((V, DP), jnp.float32),
    )(codebook, w_pad)


_sc_mesh = plsc.VectorSubcoreMesh(core_axis_name="c", subcore_axis_name="s")


@functools.partial(
    pl.kernel,
    out_type=jax.ShapeDtypeStruct((B, DP), jnp.float32),
    mesh=_sc_mesh,
    scratch_types=[
        pltpu.VMEM((4 * BPW,), jnp.int32),
        pltpu.VMEM((C, DP), jnp.float32),
        pltpu.VMEM((C, DP), jnp.float32),
        pltpu.VMEM((C, DP), jnp.float32),
        pltpu.VMEM((C, DP), jnp.float32),
        pltpu.SemaphoreType.DMA,
    ],
)
def _gather_sum(m_hbm, i1_hbm, i2_hbm, i4_hbm, i5_hbm, out_hbm,
                idx_v, acc, b2, b4, b5, sem):
    wid = lax.axis_index("s") * NC + lax.axis_index("c")
    base = wid * BPW
    pltpu.sync_copy(i1_hbm.at[pl.ds(base, BPW)], idx_v.at[pl.ds(0 * BPW, BPW)])
    pltpu.sync_copy(i2_hbm.at[pl.ds(base, BPW)], idx_v.at[pl.ds(1 * BPW, BPW)])
    pltpu.sync_copy(i4_hbm.at[pl.ds(base, BPW)], idx_v.at[pl.ds(2 * BPW, BPW)])
    pltpu.sync_copy(i5_hbm.at[pl.ds(base, BPW)], idx_v.at[pl.ds(3 * BPW, BPW)])

    for j in range(NCH):
        off = j * C
        d1 = pltpu.async_copy(m_hbm.at[idx_v.at[pl.ds(0 * BPW + off, C)]], acc, sem)
        d2 = pltpu.async_copy(m_hbm.at[idx_v.at[pl.ds(1 * BPW + off, C)]], b2, sem)
        d4 = pltpu.async_copy(m_hbm.at[idx_v.at[pl.ds(2 * BPW + off, C)]], b4, sem)
        d5 = pltpu.async_copy(m_hbm.at[idx_v.at[pl.ds(3 * BPW + off, C)]], b5, sem)
        d1.wait()
        d2.wait()
        d4.wait()
        d5.wait()

        def _row(r, carry):
            def _col(k, carry2):
                o = k * LANES
                s = (acc[r, pl.ds(o, LANES)] + b2[r, pl.ds(o, LANES)]
                     + b4[r, pl.ds(o, LANES)] + b5[r, pl.ds(o, LANES)])
                acc[r, pl.ds(o, LANES)] = s
                return carry2
            return lax.fori_loop(0, DP // LANES, _col, carry)
        lax.fori_loop(0, C, _row, 0)

        pltpu.sync_copy(acc, out_hbm.at[pl.ds(base + off, C)])


def kernel(x1, x2, x4, x5, codebook, W):
    w_pad = jnp.pad(W, ((0, DP - V), (0, 0)))
    table = _make_table(codebook, w_pad)
    out = _gather_sum(table, x1, x2, x4, x5)
    return out[:, :D]


# speedup vs baseline: 1.1935x; 1.1447x over previous
import functools

import jax
import jax.numpy as jnp
from jax import lax
from jax.experimental import pallas as pl
from jax.experimental.pallas import tpu as pltpu
from jax.experimental.pallas import tpu_sc as plsc

V = 1000
D = 1000
DP = 1024
B = 4096
NC, NS = 2, 16
NW = NC * NS
BPW = B // NW
C = 8
NCH = BPW // C
LANES = 16
G = 4 * C
NFULL = (D - LANES) // (2 * LANES)
TAIL = D - LANES


def _mm_body(a_ref, b_ref, o_ref):
    o_ref[...] = lax.dot_general(
        a_ref[...], b_ref[...],
        dimension_numbers=(((1,), (1,)), ((), ())),
        preferred_element_type=jnp.float32,
    )


def _make_table(codebook, w_pad):
    return pl.pallas_call(
        _mm_body,
        out_shape=jax.ShapeDtypeStruct((V, DP), jnp.float32),
    )(codebook, w_pad)


_sc_mesh = plsc.VectorSubcoreMesh(core_axis_name="c", subcore_axis_name="s")


@functools.partial(
    pl.kernel,
    out_type=jax.ShapeDtypeStruct((B, DP), jnp.float32),
    mesh=_sc_mesh,
    scratch_types=[
        pltpu.VMEM((NCH * G,), jnp.int32),
        pltpu.VMEM((G, DP), jnp.float32),
        pltpu.VMEM((G, DP), jnp.float32),
        pltpu.VMEM((C, DP), jnp.float32),
        pltpu.VMEM((C, DP), jnp.float32),
        pltpu.SemaphoreType.DMA,
        pltpu.SemaphoreType.DMA,
        pltpu.SemaphoreType.DMA,
        pltpu.SemaphoreType.DMA,
    ],
)
def _gather_sum(m_hbm, idx_hbm, out_hbm, idx_v, buf_a, buf_b, ob_a, ob_b,
                sg_a, sg_b, st_a, st_b):
    wid = lax.axis_index("s") * NC + lax.axis_index("c")
    base = wid * BPW

    pltpu.sync_copy(idx_hbm.at[pl.ds(wid * (NCH * G), NCH * G)], idx_v)

    bufs = (buf_a, buf_b)
    obufs = (ob_a, ob_b)
    gsems = (sg_a, sg_b)
    ssems = (st_a, st_b)

    def start_gather(j, k):
        return pltpu.async_copy(
            m_hbm.at[idx_v.at[pl.ds(j * G, G)]], bufs[k], gsems[k])

    def compute(buf, obuf):
        def one(r, o):
            s = (buf[r, pl.ds(o, LANES)]
                 + buf[C + r, pl.ds(o, LANES)]
                 + buf[2 * C + r, pl.ds(o, LANES)]
                 + buf[3 * C + r, pl.ds(o, LANES)])
            obuf[r, pl.ds(o, LANES)] = s

        def row(r, carry):
            def col(u, carry2):
                one(r, u * (2 * LANES))
                one(r, u * (2 * LANES) + LANES)
                return carry2
            lax.fori_loop(0, DP // (2 * LANES), col, 0)
            return carry
        lax.fori_loop(0, C, row, 0)

    gh = {0: start_gather(0, 0)}
    sh = {}
    for j in range(NCH):
        k = j % 2
        if j + 1 < NCH:
            gh[j + 1] = start_gather(j + 1, 1 - k)
        gh[j].wait()
        if j - 2 >= 0:
            sh[j - 2].wait()
        compute(bufs[k], obufs[k])
        sh[j] = pltpu.async_copy(
            obufs[k], out_hbm.at[pl.ds(base + j * C, C)], ssems[k])
    sh[NCH - 2].wait()
    sh[NCH - 1].wait()


def _interleave_indices(x1, x2, x4, x5):
    xs = jnp.stack([x1, x2, x4, x5])
    xs = xs.reshape(4, NW, NCH, C)
    xs = jnp.transpose(xs, (1, 2, 0, 3))
    return xs.reshape(B * 4)


def kernel(x1, x2, x4, x5, codebook, W):
    w_pad = jnp.pad(W, ((0, DP - V), (0, 0)))
    table = _make_table(codebook, w_pad)
    idx = _interleave_indices(x1, x2, x4, x5)
    return _gather_sum(table, idx)[:, :D]
